# baseline (speedup 1.0000x reference)
import functools

import jax
import jax.numpy as jnp
from jax import lax
from jax.experimental import pallas as pl
from jax.experimental.pallas import tpu as pltpu
from jax.experimental.pallas import tpu_sc as plsc

N_ATOMS = 10000
N_PAIRS = 320000
F = 128
N_RBF = 20

_LOG2 = 0.6931471805599453

_NC = 2
_NS = 16
_NW = _NC * _NS
_C = 32
_CR = _C // 4
_RPS = 624
_TAIL = N_ATOMS - _NS * _RPS

_NSPLIT = 2
_ES = N_PAIRS // _NSPLIT
_EPW_STD = 4992
_EPW_MAX = _ES - (_NW - 1) * _EPW_STD
_EB = 2000


def _ssp(x):
    return jnp.maximum(x, 0.0) + jnp.log1p(jnp.exp(-jnp.abs(x))) - _LOG2


def _dot(a, b):
    return jax.lax.dot_general(a, b, (((1,), (0,)), ((), ())),
                               preferred_element_type=jnp.float32)


def _mm(a, b):
    ah = a.astype(jnp.bfloat16)
    al = (a - ah.astype(jnp.float32)).astype(jnp.bfloat16)
    bh = b.astype(jnp.bfloat16)
    bl = (b - bh.astype(jnp.float32)).astype(jnp.bfloat16)
    return _dot(ah, bh) + _dot(ah, bl) + _dot(al, bh)


def _emb_body(a_ref, w_ref, o_ref):
    o_ref[...] = _mm(a_ref[...], w_ref[...])


def _filter_body(fb_ref, w1e_ref, bf1_ref, wf2_ref, bf2_ref, o_ref):
    fb = fb_ref[...]
    h4 = _ssp(_dot(fb.astype(jnp.bfloat16), w1e_ref[...]) + bf1_ref[...])
    for j in range(4):
        hj = h4[:, 128 * j:128 * (j + 1)]
        cut = fb[:, 32 * j + N_RBF:32 * j + N_RBF + 1]
        wj = (_dot(hj.astype(jnp.bfloat16), wf2_ref[...]) + bf2_ref[...]) * cut
        o_ref[:, 128 * j:128 * (j + 1)] = wj


def _filter_block(fflat, W1e, bf1t, Wf2b, bf2, s):
    off = s * (_ES // 4 // _EB)
    return pl.pallas_call(
        _filter_body,
        grid=(_ES // 4 // _EB,),
        in_specs=[
            pl.BlockSpec((_EB, F), lambda i: (i + off, 0)),
            pl.BlockSpec((F, 4 * F), lambda i: (0, 0)),
            pl.BlockSpec((1, 4 * F), lambda i: (0, 0)),
            pl.BlockSpec((F, F), lambda i: (0, 0)),
            pl.BlockSpec((1, F), lambda i: (0, 0)),
        ],
        out_specs=pl.BlockSpec((_EB, 4 * F), lambda i: (i, 0)),
        out_shape=jax.ShapeDtypeStruct((_ES // 4, 4 * F), jnp.float32),
    )(fflat, W1e, bf1t, Wf2b, bf2.reshape(1, F))


def _out_body(*refs):
    p_refs = refs[:-5]
    w2_ref, b2_ref, w3_ref, b3_ref, o_ref = refs[-5:]
    s = p_refs[0][...]
    for p_ref in p_refs[1:]:
        s = s + p_ref[...]
    h = _ssp(_mm(s, w2_ref[...]) + b2_ref[...])
    o_ref[...] = _mm(h, w3_ref[...]) + b3_ref[...]


def _sc_edge_kernel(emb, wij, idx_i, idx_j, base_edge):
    mesh = plsc.VectorSubcoreMesh(core_axis_name="c", subcore_axis_name="s")

    @functools.partial(
        pl.kernel,
        out_type=(jax.ShapeDtypeStruct((N_ATOMS, F), jnp.float32),
                  jax.ShapeDtypeStruct((N_ATOMS, F), jnp.float32)),
        mesh=mesh,
        scratch_types=[
            pltpu.VMEM((_EPW_MAX,), jnp.int32),
            pltpu.VMEM((_EPW_MAX,), jnp.int32),
            pltpu.VMEM((_C, F), jnp.float32),
            pltpu.VMEM((_C, F), jnp.float32),
            pltpu.VMEM((_CR, 4 * F), jnp.float32),
            pltpu.VMEM((_CR, 4 * F), jnp.float32),
            pltpu.VMEM((_C, F), jnp.float32),
            pltpu.VMEM((_C, F), jnp.float32),
            pltpu.VMEM_SHARED((N_ATOMS, F), jnp.float32),
            pltpu.SemaphoreType.DMA,
            pltpu.SemaphoreType.DMA,
            pltpu.SemaphoreType.DMA,
            pltpu.SemaphoreType.DMA,
            pltpu.SemaphoreType.DMA,
            pltpu.SemaphoreType.DMA,
        ],
    )
    def k(emb_hbm, wij_hbm, idxi_hbm, idxj_hbm, out0_hbm, out1_hbm,
          idxj_t, idxi_t, xj0, xj1, w0, w1, pr0, pr1, acc,
          sg0, sg1, sw0, sw1, ss0, ss1):
        xj = (xj0, xj1)
        wv = (w0, w1)
        pr = (pr0, pr1)
        sg = (sg0, sg1)
        sw = (sw0, sw1)
        ss = (ss0, ss1)

        cid = lax.axis_index("c")
        sid = lax.axis_index("s")
        wid = cid * _NS + sid
        base0 = wid * _EPW_STD
        rowb = wid * (_EPW_STD // 4)
        nch = 156 + 8 * (wid == _NW - 1).astype(jnp.int32)

        pltpu.sync_copy(idxj_hbm.at[pl.ds(base_edge + base0, _EPW_MAX)], idxj_t)
        pltpu.sync_copy(idxi_hbm.at[pl.ds(base_edge + base0, _EPW_MAX)], idxi_t)

        def issue_loads(c, p):
            pltpu.async_copy(emb_hbm.at[idxj_t.at[pl.ds(c * _C, _C)]],
                             xj[p], sg[p])
            pltpu.async_copy(
                wij_hbm.at[pl.ds(rowb + c * _CR, _CR)], wv[p], sw[p])

        issue_loads(0, 0)
        issue_loads(1, 1)

        @pl.loop(0, _C)
        def _(r):
            for c in range(F // 16):
                pr0[r, pl.ds(c * 16, 16)] = jnp.zeros((16,), jnp.float32)

        @pl.loop(0, _RPS - _C, step=_C)
        def _(r0):
            pltpu.sync_copy(pr0, acc.at[pl.ds(sid * _RPS + r0, _C)])

        rem = _RPS % _C if _RPS % _C else _C
        pltpu.sync_copy(pr0.at[pl.ds(0, rem)],
                        acc.at[pl.ds(sid * _RPS + _RPS - rem, rem)])

        @pl.when(sid == 0)
        def _():
            pltpu.sync_copy(pr0.at[pl.ds(0, _TAIL)],
                            acc.at[pl.ds(_NS * _RPS, _TAIL)])

        plsc.subcore_barrier()

        def process(c, p):
            pltpu.make_async_copy(
                emb_hbm.at[idxj_t.at[pl.ds(c * _C, _C)]], xj[p], sg[p]).wait()
            pltpu.make_async_copy(
                wij_hbm.at[pl.ds(rowb + c * _CR, _CR)], wv[p], sw[p]).wait()

            @pl.when(c >= 2)
            def _():
                pltpu.make_async_copy(
                    pr[p], acc.at[idxi_t.at[pl.ds(c * _C, _C)]], ss[p]).wait()

            @pl.loop(0, _CR)
            def _(r):
                for j in range(4):
                    for col in range(F // 16):
                        sx = pl.ds(col * 16, 16)
                        sw_ = pl.ds(128 * j + col * 16, 16)
                        pr[p][4 * r + j, sx] = xj[p][4 * r + j, sx] * wv[p][r, sw_]

            pltpu.async_copy(pr[p], acc.at[idxi_t.at[pl.ds(c * _C, _C)]],
                             ss[p], add=True)

            @pl.when(c + 2 < nch)
            def _():
                issue_loads(c + 2, p)

        @pl.loop(0, nch, step=2)
        def _(ch):
            process(ch, 0)
            process(ch + 1, 1)

        pltpu.make_async_copy(pr[0], acc.at[idxi_t.at[pl.ds(0, _C)]], ss[0]).wait()
        pltpu.make_async_copy(pr[1], acc.at[idxi_t.at[pl.ds(0, _C)]], ss[1]).wait()

        plsc.subcore_barrier()

        def dump(out_hbm):
            pltpu.sync_copy(acc.at[pl.ds(sid * _RPS, _RPS)],
                            out_hbm.at[pl.ds(sid * _RPS, _RPS)])

            @pl.when(sid == 0)
            def _():
                pltpu.sync_copy(acc.at[pl.ds(_NS * _RPS, _TAIL)],
                                out_hbm.at[pl.ds(_NS * _RPS, _TAIL)])

        @pl.when(cid == 0)
        def _():
            dump(out0_hbm)

        @pl.when(cid == 1)
        def _():
            dump(out1_hbm)

    return k(emb, wij, idx_i, idx_j)


def kernel(atomic_embedding, pair_indices, f_ij, f_ij_cutoff,
           W_in, Wf1, bf1, Wf2, bf2, W2, b2, W3, b3):
    idx_i = pair_indices[0]
    idx_j = pair_indices[1]

    fpad = jnp.concatenate(
        [f_ij, f_ij_cutoff,
         jnp.zeros((N_PAIRS, 32 - N_RBF - 1), jnp.float32)], axis=1)
    fflat = fpad.reshape(N_PAIRS // 4, F)

    W1p = jnp.pad(Wf1, ((0, 32 - N_RBF), (0, 0)))
    W1e = jax.scipy.linalg.block_diag(W1p, W1p, W1p, W1p)
    W1e = W1e.astype(jnp.bfloat16)
    bf1t = jnp.tile(bf1.reshape(1, F), (1, 4))
    Wf2b = Wf2.astype(jnp.bfloat16)

    emb = pl.pallas_call(
        _emb_body,
        out_shape=jax.ShapeDtypeStruct((N_ATOMS, F), jnp.float32),
    )(atomic_embedding, W_in)

    partials = []
    for s in range(_NSPLIT):
        wij = _filter_block(fflat, W1e, bf1t, Wf2b, bf2, s)
        partials.extend(_sc_edge_kernel(emb, wij, idx_i, idx_j, s * _ES))

    NB = 2000
    out = pl.pallas_call(
        _out_body,
        grid=(N_ATOMS // NB,),
        in_specs=[pl.BlockSpec((NB, F), lambda i: (i, 0))] * len(partials) + [
            pl.BlockSpec((F, F), lambda i: (0, 0)),
            pl.BlockSpec((1, F), lambda i: (0, 0)),
            pl.BlockSpec((F, F), lambda i: (0, 0)),
            pl.BlockSpec((1, F), lambda i: (0, 0)),
        ],
        out_specs=pl.BlockSpec((NB, F), lambda i: (i, 0)),
        out_shape=jax.ShapeDtypeStruct((N_ATOMS, F), jnp.float32),
    )(*partials, W2, b2.reshape(1, F), W3, b3.reshape(1, F))

    return out


# speedup vs baseline: 1.2328x; 1.2328x over previous
import functools

import jax
import jax.numpy as jnp
from jax import lax
from jax.experimental import pallas as pl
from jax.experimental.pallas import tpu as pltpu
from jax.experimental.pallas import tpu_sc as plsc

N_ATOMS = 10000
N_PAIRS = 320000
F = 128
N_RBF = 20

_LOG2 = 0.6931471805599453

_NC = 2
_NS = 16
_NW = _NC * _NS
_C = 40
_RPS = 624
_TAIL = N_ATOMS - _NS * _RPS

_NSPLIT = 2
_ES = N_PAIRS // _NSPLIT
_EPW = _ES // _NW
_EB = 8000


def _ssp(x):
    return jnp.maximum(x, 0.0) + jnp.log1p(jnp.exp(-jnp.abs(x))) - _LOG2


def _dot(a, b):
    return jax.lax.dot_general(a, b, (((1,), (0,)), ((), ())),
                               preferred_element_type=jnp.float32)


def _mm(a, b):
    ah = a.astype(jnp.bfloat16)
    al = (a - ah.astype(jnp.float32)).astype(jnp.bfloat16)
    bh = b.astype(jnp.bfloat16)
    bl = (b - bh.astype(jnp.float32)).astype(jnp.bfloat16)
    return _dot(ah, bh) + _dot(ah, bl) + _dot(al, bh)


def _emb_body(a_ref, w_ref, o_ref):
    o_ref[...] = _mm(a_ref[...], w_ref[...])


def _filter_body(f_ref, cut_ref, wf1_ref, bf1_ref, wf2_ref, bf2_ref, o_ref):
    h = _ssp(_mm(f_ref[...], wf1_ref[...]) + bf1_ref[...])
    o_ref[...] = (_dot(h.astype(jnp.bfloat16), wf2_ref[...].astype(jnp.bfloat16))
                  + bf2_ref[...]) * cut_ref[...]


def _filter_block(f_ij, f_ij_cutoff, Wf1, bf1, Wf2, bf2, s):
    off = s * (_ES // _EB)
    return pl.pallas_call(
        _filter_body,
        grid=(_ES // _EB,),
        in_specs=[
            pl.BlockSpec((_EB, N_RBF), lambda i: (i + off, 0)),
            pl.BlockSpec((_EB, 1), lambda i: (i + off, 0)),
            pl.BlockSpec((N_RBF, F), lambda i: (0, 0)),
            pl.BlockSpec((1, F), lambda i: (0, 0)),
            pl.BlockSpec((F, F), lambda i: (0, 0)),
            pl.BlockSpec((1, F), lambda i: (0, 0)),
        ],
        out_specs=pl.BlockSpec((_EB, F), lambda i: (i, 0)),
        out_shape=jax.ShapeDtypeStruct((_ES, F), jnp.float32),
    )(f_ij, f_ij_cutoff, Wf1, bf1.reshape(1, F), Wf2, bf2.reshape(1, F))


def _out_body(*refs):
    p_refs = refs[:-5]
    w2_ref, b2_ref, w3_ref, b3_ref, o_ref = refs[-5:]
    s = p_refs[0][...]
    for p_ref in p_refs[1:]:
        s = s + p_ref[...]
    h = _ssp(_mm(s, w2_ref[...]) + b2_ref[...])
    o_ref[...] = _mm(h, w3_ref[...]) + b3_ref[...]


def _sc_edge_kernel(emb, wij, idx_i, idx_j, base_edge):
    epw = _EPW
    nch = epw // _C
    mesh = plsc.VectorSubcoreMesh(core_axis_name="c", subcore_axis_name="s")

    @functools.partial(
        pl.kernel,
        out_type=(jax.ShapeDtypeStruct((N_ATOMS, F), jnp.float32),
                  jax.ShapeDtypeStruct((N_ATOMS, F), jnp.float32)),
        mesh=mesh,
        scratch_types=[
            pltpu.VMEM((epw,), jnp.int32),
            pltpu.VMEM((epw,), jnp.int32),
            pltpu.VMEM((_C, F), jnp.float32),
            pltpu.VMEM((_C, F), jnp.float32),
            pltpu.VMEM((_C, F), jnp.float32),
            pltpu.VMEM((_C, F), jnp.float32),
            pltpu.VMEM((_C, F), jnp.float32),
            pltpu.VMEM((_C, F), jnp.float32),
            pltpu.VMEM_SHARED((N_ATOMS, F), jnp.float32),
            pltpu.SemaphoreType.DMA,
            pltpu.SemaphoreType.DMA,
            pltpu.SemaphoreType.DMA,
            pltpu.SemaphoreType.DMA,
            pltpu.SemaphoreType.DMA,
            pltpu.SemaphoreType.DMA,
        ],
    )
    def k(emb_hbm, wij_hbm, idxi_hbm, idxj_hbm, out0_hbm, out1_hbm,
          idxj_t, idxi_t, xj0, xj1, w0, w1, pr0, pr1, acc,
          sg0, sg1, sw0, sw1, ss0, ss1):
        xj = (xj0, xj1)
        wv = (w0, w1)
        pr = (pr0, pr1)
        sg = (sg0, sg1)
        sw = (sw0, sw1)
        ss = (ss0, ss1)

        cid = lax.axis_index("c")
        sid = lax.axis_index("s")
        wid = cid * _NS + sid
        base0 = wid * epw

        pltpu.sync_copy(idxj_hbm.at[pl.ds(base_edge + base0, epw)], idxj_t)
        pltpu.sync_copy(idxi_hbm.at[pl.ds(base_edge + base0, epw)], idxi_t)

        def issue_loads(c, p):
            pltpu.async_copy(emb_hbm.at[idxj_t.at[pl.ds(c * _C, _C)]],
                             xj[p], sg[p])
            pltpu.async_copy(wij_hbm.at[pl.ds(base0 + c * _C, _C)], wv[p], sw[p])

        issue_loads(0, 0)
        issue_loads(1, 1)

        @pl.loop(0, _C)
        def _(r):
            for c in range(F // 16):
                pr0[r, pl.ds(c * 16, 16)] = jnp.zeros((16,), jnp.float32)

        @pl.loop(0, _RPS - _C, step=_C)
        def _(r0):
            pltpu.sync_copy(pr0, acc.at[pl.ds(sid * _RPS + r0, _C)])

        rem = _RPS % _C if _RPS % _C else _C
        pltpu.sync_copy(pr0.at[pl.ds(0, rem)],
                        acc.at[pl.ds(sid * _RPS + _RPS - rem, rem)])

        @pl.when(sid == 0)
        def _():
            pltpu.sync_copy(pr0.at[pl.ds(0, _TAIL)],
                            acc.at[pl.ds(_NS * _RPS, _TAIL)])

        plsc.subcore_barrier()

        def process(c, p, prefetch):
            pltpu.make_async_copy(
                emb_hbm.at[idxj_t.at[pl.ds(c * _C, _C)]], xj[p], sg[p]).wait()
            pltpu.make_async_copy(
                wij_hbm.at[pl.ds(base0 + c * _C, _C)], wv[p], sw[p]).wait()

            @pl.when(c >= 2)
            def _():
                pltpu.make_async_copy(
                    pr[p], acc.at[idxi_t.at[pl.ds(c * _C, _C)]], ss[p]).wait()

            @pl.loop(0, _C)
            def _(e):
                for col in range(F // 16):
                    s = pl.ds(col * 16, 16)
                    pr[p][e, s] = xj[p][e, s] * wv[p][e, s]

            pltpu.async_copy(pr[p], acc.at[idxi_t.at[pl.ds(c * _C, _C)]],
                             ss[p], add=True)
            if prefetch:
                @pl.when(c + 2 < nch)
                def _():
                    issue_loads(c + 2, p)

        @pl.loop(0, nch - (nch % 2), step=2)
        def _(ch):
            process(ch, 0, True)
            process(ch + 1, 1, True)

        if nch % 2:
            process(nch - 1, 0, False)

        pltpu.make_async_copy(pr[0], acc.at[idxi_t.at[pl.ds(0, _C)]], ss[0]).wait()
        pltpu.make_async_copy(pr[1], acc.at[idxi_t.at[pl.ds(0, _C)]], ss[1]).wait()

        plsc.subcore_barrier()

        def dump(out_hbm):
            pltpu.sync_copy(acc.at[pl.ds(sid * _RPS, _RPS)],
                            out_hbm.at[pl.ds(sid * _RPS, _RPS)])

            @pl.when(sid == 0)
            def _():
                pltpu.sync_copy(acc.at[pl.ds(_NS * _RPS, _TAIL)],
                                out_hbm.at[pl.ds(_NS * _RPS, _TAIL)])

        @pl.when(cid == 0)
        def _():
            dump(out0_hbm)

        @pl.when(cid == 1)
        def _():
            dump(out1_hbm)

    return k(emb, wij, idx_i, idx_j)


def kernel(atomic_embedding, pair_indices, f_ij, f_ij_cutoff,
           W_in, Wf1, bf1, Wf2, bf2, W2, b2, W3, b3):
    idx_i = pair_indices[0]
    idx_j = pair_indices[1]

    emb = pl.pallas_call(
        _emb_body,
        out_shape=jax.ShapeDtypeStruct((N_ATOMS, F), jnp.float32),
    )(atomic_embedding, W_in)

    partials = []
    for s in range(_NSPLIT):
        wij = _filter_block(f_ij, f_ij_cutoff, Wf1, bf1, Wf2, bf2, s)
        partials.extend(_sc_edge_kernel(emb, wij, idx_i, idx_j, s * _ES))

    NB = 2000
    out = pl.pallas_call(
        _out_body,
        grid=(N_ATOMS // NB,),
        in_specs=[pl.BlockSpec((NB, F), lambda i: (i, 0))] * len(partials) + [
            pl.BlockSpec((F, F), lambda i: (0, 0)),
            pl.BlockSpec((1, F), lambda i: (0, 0)),
            pl.BlockSpec((F, F), lambda i: (0, 0)),
            pl.BlockSpec((1, F), lambda i: (0, 0)),
        ],
        out_specs=pl.BlockSpec((NB, F), lambda i: (i, 0)),
        out_shape=jax.ShapeDtypeStruct((N_ATOMS, F), jnp.float32),
    )(*partials, W2, b2.reshape(1, F), W3, b3.reshape(1, F))

    return out


# speedup vs baseline: 1.4244x; 1.1554x over previous
import functools

import jax
import jax.numpy as jnp
from jax import lax
from jax.experimental import pallas as pl
from jax.experimental.pallas import tpu as pltpu
from jax.experimental.pallas import tpu_sc as plsc

N_ATOMS = 10000
N_PAIRS = 320000
F = 128
N_RBF = 20

_LOG2 = 0.6931471805599453

_NC = 2
_NS = 16
_NW = _NC * _NS
_C = 40
_RPS = 624
_TAIL = N_ATOMS - _NS * _RPS

_NSPLIT = 2
_ES = N_PAIRS // _NSPLIT
_EPW = _ES // _NW
_EB = 16000


def _ssp(x):
    return jnp.maximum(x, 0.0) + jnp.log1p(jnp.exp(-jnp.abs(x))) - _LOG2


def _dot(a, b):
    return jax.lax.dot_general(a, b, (((1,), (0,)), ((), ())),
                               preferred_element_type=jnp.float32)


def _mm(a, b):
    ah = a.astype(jnp.bfloat16)
    al = (a - ah.astype(jnp.float32)).astype(jnp.bfloat16)
    bh = b.astype(jnp.bfloat16)
    bl = (b - bh.astype(jnp.float32)).astype(jnp.bfloat16)
    return _dot(ah, bh) + _dot(ah, bl) + _dot(al, bh)


def _emb_body(a_ref, w_ref, o_ref):
    o_ref[...] = _mm(a_ref[...], w_ref[...])


def _make_filter_body(off):
    cb = _EB // 128

    def _filter_body(f_ref, cut_ref, wf1_ref, bf1_ref, wf2_ref, bf2_ref, o_ref):
        h = _ssp(_mm(f_ref[...], wf1_ref[...]) + bf1_ref[...])
        w = (_dot(h.astype(jnp.bfloat16), wf2_ref[...].astype(jnp.bfloat16))
             + bf2_ref[...])
        cut = cut_ref[pl.ds((pl.program_id(0) + off) * cb, cb), :]
        w3 = w.reshape(cb, 128, F) * cut[:, :, None]
        o_ref[...] = w3.reshape(_EB, F)

    return _filter_body


def _filter_block(f_ij, cutr, Wf1, bf1, Wf2, bf2, s):
    off = s * (_ES // _EB)
    return pl.pallas_call(
        _make_filter_body(off),
        grid=(_ES // _EB,),
        in_specs=[
            pl.BlockSpec((_EB, N_RBF), lambda i: (i + off, 0)),
            pl.BlockSpec((N_PAIRS // 128, 128), lambda i: (0, 0)),
            pl.BlockSpec((N_RBF, F), lambda i: (0, 0)),
            pl.BlockSpec((1, F), lambda i: (0, 0)),
            pl.BlockSpec((F, F), lambda i: (0, 0)),
            pl.BlockSpec((1, F), lambda i: (0, 0)),
        ],
        out_specs=pl.BlockSpec((_EB, F), lambda i: (i, 0)),
        out_shape=jax.ShapeDtypeStruct((_ES, F), jnp.float32),
    )(f_ij, cutr, Wf1, bf1.reshape(1, F), Wf2, bf2.reshape(1, F))


def _out_body(*refs):
    p_refs = refs[:-5]
    w2_ref, b2_ref, w3_ref, b3_ref, o_ref = refs[-5:]
    s = p_refs[0][...]
    for p_ref in p_refs[1:]:
        s = s + p_ref[...]
    h = _ssp(_mm(s, w2_ref[...]) + b2_ref[...])
    o_ref[...] = _mm(h, w3_ref[...]) + b3_ref[...]


def _sc_edge_kernel(emb, wij, idx_i, idx_j, base_edge):
    epw = _EPW
    nch = epw // _C
    mesh = plsc.VectorSubcoreMesh(core_axis_name="c", subcore_axis_name="s")

    @functools.partial(
        pl.kernel,
        out_type=(jax.ShapeDtypeStruct((N_ATOMS, F), jnp.float32),
                  jax.ShapeDtypeStruct((N_ATOMS, F), jnp.float32)),
        mesh=mesh,
        scratch_types=[
            pltpu.VMEM((epw,), jnp.int32),
            pltpu.VMEM((epw,), jnp.int32),
            pltpu.VMEM((_C, F), jnp.float32),
            pltpu.VMEM((_C, F), jnp.float32),
            pltpu.VMEM((_C, F), jnp.float32),
            pltpu.VMEM((_C, F), jnp.float32),
            pltpu.VMEM((_C, F), jnp.float32),
            pltpu.VMEM((_C, F), jnp.float32),
            pltpu.VMEM_SHARED((N_ATOMS, F), jnp.float32),
            pltpu.SemaphoreType.DMA,
            pltpu.SemaphoreType.DMA,
            pltpu.SemaphoreType.DMA,
            pltpu.SemaphoreType.DMA,
            pltpu.SemaphoreType.DMA,
            pltpu.SemaphoreType.DMA,
        ],
    )
    def k(emb_hbm, wij_hbm, idxi_hbm, idxj_hbm, out0_hbm, out1_hbm,
          idxj_t, idxi_t, xj0, xj1, w0, w1, pr0, pr1, acc,
          sg0, sg1, sw0, sw1, ss0, ss1):
        xj = (xj0, xj1)
        wv = (w0, w1)
        pr = (pr0, pr1)
        sg = (sg0, sg1)
        sw = (sw0, sw1)
        ss = (ss0, ss1)

        cid = lax.axis_index("c")
        sid = lax.axis_index("s")
        wid = cid * _NS + sid
        base0 = wid * epw

        pltpu.sync_copy(idxj_hbm.at[pl.ds(base_edge + base0, epw)], idxj_t)
        pltpu.sync_copy(idxi_hbm.at[pl.ds(base_edge + base0, epw)], idxi_t)

        def issue_loads(c, p):
            pltpu.async_copy(emb_hbm.at[idxj_t.at[pl.ds(c * _C, _C)]],
                             xj[p], sg[p])
            pltpu.async_copy(wij_hbm.at[pl.ds(base0 + c * _C, _C)], wv[p], sw[p])

        issue_loads(0, 0)
        issue_loads(1, 1)

        @pl.loop(0, _C)
        def _(r):
            for c in range(F // 16):
                pr0[r, pl.ds(c * 16, 16)] = jnp.zeros((16,), jnp.float32)

        @pl.loop(0, _RPS - _C, step=_C)
        def _(r0):
            pltpu.sync_copy(pr0, acc.at[pl.ds(sid * _RPS + r0, _C)])

        rem = _RPS % _C if _RPS % _C else _C
        pltpu.sync_copy(pr0.at[pl.ds(0, rem)],
                        acc.at[pl.ds(sid * _RPS + _RPS - rem, rem)])

        @pl.when(sid == 0)
        def _():
            pltpu.sync_copy(pr0.at[pl.ds(0, _TAIL)],
                            acc.at[pl.ds(_NS * _RPS, _TAIL)])

        plsc.subcore_barrier()

        def process(c, p, prefetch):
            pltpu.make_async_copy(
                emb_hbm.at[idxj_t.at[pl.ds(c * _C, _C)]], xj[p], sg[p]).wait()
            pltpu.make_async_copy(
                wij_hbm.at[pl.ds(base0 + c * _C, _C)], wv[p], sw[p]).wait()

            @pl.when(c >= 2)
            def _():
                pltpu.make_async_copy(
                    pr[p], acc.at[idxi_t.at[pl.ds(c * _C, _C)]], ss[p]).wait()

            @pl.loop(0, _C)
            def _(e):
                for col in range(F // 16):
                    s = pl.ds(col * 16, 16)
                    pr[p][e, s] = xj[p][e, s] * wv[p][e, s]

            pltpu.async_copy(pr[p], acc.at[idxi_t.at[pl.ds(c * _C, _C)]],
                             ss[p], add=True)
            if prefetch:
                @pl.when(c + 2 < nch)
                def _():
                    issue_loads(c + 2, p)

        @pl.loop(0, nch - (nch % 2), step=2)
        def _(ch):
            process(ch, 0, True)
            process(ch + 1, 1, True)

        if nch % 2:
            process(nch - 1, 0, False)

        pltpu.make_async_copy(pr[0], acc.at[idxi_t.at[pl.ds(0, _C)]], ss[0]).wait()
        pltpu.make_async_copy(pr[1], acc.at[idxi_t.at[pl.ds(0, _C)]], ss[1]).wait()

        plsc.subcore_barrier()

        def dump(out_hbm):
            pltpu.sync_copy(acc.at[pl.ds(sid * _RPS, _RPS)],
                            out_hbm.at[pl.ds(sid * _RPS, _RPS)])

            @pl.when(sid == 0)
            def _():
                pltpu.sync_copy(acc.at[pl.ds(_NS * _RPS, _TAIL)],
                                out_hbm.at[pl.ds(_NS * _RPS, _TAIL)])

        @pl.when(cid == 0)
        def _():
            dump(out0_hbm)

        @pl.when(cid == 1)
        def _():
            dump(out1_hbm)

    return k(emb, wij, idx_i, idx_j)


def kernel(atomic_embedding, pair_indices, f_ij, f_ij_cutoff,
           W_in, Wf1, bf1, Wf2, bf2, W2, b2, W3, b3):
    idx_i = pair_indices[0]
    idx_j = pair_indices[1]

    emb = pl.pallas_call(
        _emb_body,
        out_shape=jax.ShapeDtypeStruct((N_ATOMS, F), jnp.float32),
    )(atomic_embedding, W_in)

    cutr = f_ij_cutoff.reshape(N_PAIRS // 128, 128)
    partials = []
    for s in range(_NSPLIT):
        wij = _filter_block(f_ij, cutr, Wf1, bf1, Wf2, bf2, s)
        partials.extend(_sc_edge_kernel(emb, wij, idx_i, idx_j, s * _ES))

    NB = 2000
    out = pl.pallas_call(
        _out_body,
        grid=(N_ATOMS // NB,),
        in_specs=[pl.BlockSpec((NB, F), lambda i: (i, 0))] * len(partials) + [
            pl.BlockSpec((F, F), lambda i: (0, 0)),
            pl.BlockSpec((1, F), lambda i: (0, 0)),
            pl.BlockSpec((F, F), lambda i: (0, 0)),
            pl.BlockSpec((1, F), lambda i: (0, 0)),
        ],
        out_specs=pl.BlockSpec((NB, F), lambda i: (i, 0)),
        out_shape=jax.ShapeDtypeStruct((N_ATOMS, F), jnp.float32),
    )(*partials, W2, b2.reshape(1, F), W3, b3.reshape(1, F))

    return out
